# Initial kernel scaffold; baseline (speedup 1.0000x reference)
import jax
import jax.numpy as jnp
from jax.experimental import pallas as pl


def kernel(E, V, u, r, s, We, be, Wn, bn, Wg, bg):
    raise NotImplementedError("write your pallas kernel here")


# speedup vs baseline: 28.0720x; 28.0720x over previous
import functools

import jax
import jax.numpy as jnp
from jax import lax
from jax.experimental import pallas as pl
from jax.experimental.pallas import tpu as pltpu
from jax.experimental.pallas import tpu_sc as plsc

Ne = 320000
Nn = 10000
NDIM = 128
NC = 2
NS = 16
NW = NC * NS
CH = Ne // NW
LANES = 16


def _ab_body(w_ref, v_ref, ab_ref):
    ab_ref[...] = lax.dot_general(
        w_ref[...], v_ref[...], (((0,), (1,)), ((), ())),
        preferred_element_type=jnp.float32)


def _sc_edge_body(r_hbm, s_hbm, e_hbm, ab_hbm, sc_hbm,
                  ep_hbm, eagg_hbm, deg_hbm,
                  r_v, s_v, e_v, a_v, b_v, ep_v, eagg_v, deg_v, sc_v):
    cid = lax.axis_index("c")
    sid = lax.axis_index("s")
    wid = sid * NC + cid
    base = wid * CH

    pltpu.sync_copy(r_hbm.at[pl.ds(base, CH)], r_v)
    pltpu.sync_copy(s_hbm.at[pl.ds(base, CH)], s_v)
    pltpu.sync_copy(e_hbm.at[pl.ds(base, CH)], e_v)
    pltpu.sync_copy(ab_hbm.at[0], a_v)
    pltpu.sync_copy(ab_hbm.at[1], b_v)
    pltpu.sync_copy(sc_hbm, sc_v)

    zeros = jnp.zeros((LANES,), jnp.float32)

    def zero_body(i, carry):
        eagg_v[pl.ds(i * LANES, LANES)] = zeros
        deg_v[pl.ds(i * LANES, LANES)] = zeros
        return carry

    lax.fori_loop(0, Nn // LANES, zero_body, 0)

    w0v = sc_v[0]
    cv = sc_v[1]
    ones = jnp.full((LANES,), 1.0, jnp.float32)

    def body(i, carry):
        off = i * LANES
        ir = r_v[pl.ds(off, LANES)]
        isv = s_v[pl.ds(off, LANES)]
        ev = e_v[pl.ds(off, LANES)]
        av = plsc.load_gather(a_v, [ir])
        bv = plsc.load_gather(b_v, [isv])
        ep = jnp.maximum(ev * w0v + av + bv + cv, 0.0)
        ep_v[pl.ds(off, LANES)] = ep
        plsc.addupdate_scatter(eagg_v, [ir], ep)
        plsc.addupdate_scatter(deg_v, [ir], ones)
        return carry

    lax.fori_loop(0, CH // LANES, body, 0)

    pltpu.sync_copy(ep_v, ep_hbm.at[pl.ds(base, CH)])
    pltpu.sync_copy(eagg_v, eagg_hbm.at[wid])
    pltpu.sync_copy(deg_v, deg_hbm.at[wid])


def _node_body(v_ref, eaggp_ref, degp_ref, wn1_ref, wn0b_ref, ncst_ref,
               wgv_ref, cst_ref, vp_ref, up_ref):
    dn = (((1,), (1,)), ((), ()))
    p = lax.dot_general(v_ref[...], wn1_ref[...], dn,
                        preferred_element_type=jnp.float32)
    d0 = (((0,), (0,)), ((), ()))
    eagg_b = lax.dot_general(eaggp_ref[...], wn0b_ref[...], d0,
                             preferred_element_type=jnp.float32)
    deg_b = lax.dot_general(degp_ref[...], jnp.ones((NW, NDIM), jnp.float32),
                            d0, preferred_element_type=jnp.float32)
    v_new = jnp.maximum(p + eagg_b + ncst_ref[...], 0.0)
    v_p = jnp.where(deg_b > 0.0, v_new, 0.0)
    vp_ref[...] = v_p
    v_bar = jnp.sum(v_p, axis=0) / Nn
    e_bar = jnp.sum(eaggp_ref[...]) / Ne
    g_dot = jnp.sum(wgv_ref[0] * v_bar)
    wg0 = cst_ref[0]
    gk = cst_ref[1]
    up_ref[...] = jnp.maximum(wg0 * e_bar + g_dot + gk, 0.0).reshape(1, 1)


def kernel(E, V, u, r, s, We, be, Wn, bn, Wg, bg):
    u0 = u[0]
    w0 = We[0, 0]
    wrs = jnp.stack([We[0, 1:1 + NDIM], We[0, 1 + NDIM:1 + 2 * NDIM]], axis=1)
    c = We[0, 1 + 2 * NDIM] * u0 + be[0]

    ab = pl.pallas_call(
        _ab_body,
        out_shape=jax.ShapeDtypeStruct((2, Nn), jnp.float32),
    )(wrs, V)

    scvec = jnp.stack([jnp.full((LANES,), w0, jnp.float32),
                       jnp.full((LANES,), c, jnp.float32)])

    mesh = plsc.VectorSubcoreMesh(core_axis_name="c", subcore_axis_name="s")
    edge_fn = pl.kernel(
        _sc_edge_body,
        mesh=mesh,
        compiler_params=pltpu.CompilerParams(needs_layout_passes=False),
        out_type=[
            jax.ShapeDtypeStruct((Ne,), jnp.float32),
            jax.ShapeDtypeStruct((NW, Nn), jnp.float32),
            jax.ShapeDtypeStruct((NW, Nn), jnp.float32),
        ],
        scratch_types=[
            pltpu.VMEM((CH,), jnp.int32),
            pltpu.VMEM((CH,), jnp.int32),
            pltpu.VMEM((CH,), jnp.float32),
            pltpu.VMEM((Nn,), jnp.float32),
            pltpu.VMEM((Nn,), jnp.float32),
            pltpu.VMEM((CH,), jnp.float32),
            pltpu.VMEM((Nn,), jnp.float32),
            pltpu.VMEM((Nn,), jnp.float32),
            pltpu.VMEM((2, LANES), jnp.float32),
        ],
    )
    ep, eagg_p, deg_p = edge_fn(r, s, E[:, 0], ab, scvec)

    wn0b = jnp.broadcast_to(Wn[:, 0][None, :], (NW, NDIM))
    wn1 = Wn[:, 1:1 + NDIM]
    ncst = (u0 * Wn[:, 1 + NDIM] + bn)[None, :]
    wgv = Wg[:, 1:1 + NDIM]
    cst = jnp.stack([Wg[0, 0], Wg[0, 1 + NDIM] * u0 + bg[0]])

    v_prime, up = pl.pallas_call(
        _node_body,
        out_shape=[
            jax.ShapeDtypeStruct((Nn, NDIM), jnp.float32),
            jax.ShapeDtypeStruct((1, 1), jnp.float32),
        ],
        in_specs=[
            pl.BlockSpec(memory_space=pltpu.VMEM),
            pl.BlockSpec(memory_space=pltpu.VMEM),
            pl.BlockSpec(memory_space=pltpu.VMEM),
            pl.BlockSpec(memory_space=pltpu.VMEM),
            pl.BlockSpec(memory_space=pltpu.VMEM),
            pl.BlockSpec(memory_space=pltpu.VMEM),
            pl.BlockSpec(memory_space=pltpu.VMEM),
            pl.BlockSpec(memory_space=pltpu.SMEM),
        ],
    )(V, eagg_p, deg_p, wn1, wn0b, ncst, wgv, cst)

    return ep[:, None], v_prime, up.reshape(1)
